# Initial kernel scaffold; baseline (speedup 1.0000x reference)
import jax
import jax.numpy as jnp
from jax.experimental import pallas as pl


def kernel(input_ids):
    raise NotImplementedError("write your pallas kernel here")


# speedup vs baseline: 15.7366x; 15.7366x over previous
import jax
import jax.numpy as jnp
from jax.experimental import pallas as pl

VOCAB_SIZE = 8192
ROWS_PER_BLOCK = 256


def _onehot_body(ids_ref, out_ref):
    ids = ids_ref[0, 0, :]
    idx = jax.lax.rem(ids, VOCAB_SIZE)
    iota = jax.lax.broadcasted_iota(jnp.int32, (ROWS_PER_BLOCK, VOCAB_SIZE), 1)
    out_ref[...] = jnp.where(iota == idx[:, None], 5.0, 0.0)


def kernel(input_ids):
    bs, seq = input_ids.shape
    n_rows = bs * seq
    grid = n_rows // ROWS_PER_BLOCK
    ids3 = input_ids.reshape(grid, 1, ROWS_PER_BLOCK)
    out = pl.pallas_call(
        _onehot_body,
        grid=(grid,),
        in_specs=[pl.BlockSpec((1, 1, ROWS_PER_BLOCK), lambda i: (i, 0, 0))],
        out_specs=pl.BlockSpec((ROWS_PER_BLOCK, VOCAB_SIZE), lambda i: (i, 0)),
        out_shape=jax.ShapeDtypeStruct((n_rows, VOCAB_SIZE), jnp.float32),
    )(ids3)
    return out.reshape(bs, seq, VOCAB_SIZE)
